# Initial kernel scaffold; baseline (speedup 1.0000x reference)
import jax
import jax.numpy as jnp
from jax.experimental import pallas as pl


def kernel(x, edge_index, c1_W1, c1_b1, c1_s, c1_be, c1_W2, c1_b2, c2_W1, c2_b1, c2_s, c2_be, c2_W2, c2_b2, m_W1, m_b1, m_W2, m_b2):
    raise NotImplementedError("write your pallas kernel here")


# speedup vs baseline: 4.2648x; 4.2648x over previous
import functools

import jax
import jax.numpy as jnp
from jax import lax
from jax.experimental import pallas as pl
from jax.experimental.pallas import tpu as pltpu
from jax.experimental.pallas import tpu_sc as plsc

N = 10000
E = 320000
D = 128
H = 128
C = 64

NC = 2
NS = 16
NW = NC * NS
CHUNK = 128
NCHUNK = -(-E // (NW * CHUNK))
EPAD = NW * NCHUNK * CHUNK
NACC = 10112
RPT = NACC // NS


def _agg_body(x_hbm, src_hbm, dst_hbm, zero_hbm, out_hbm,
              src_v, dst_v, rows_v, acc, sem):
    c = lax.axis_index("c")
    s = lax.axis_index("s")
    wid = s * NC + c

    pltpu.sync_copy(zero_hbm, acc.at[pl.ds(s * RPT, RPT)])
    pltpu.sync_copy(src_hbm.at[wid], src_v)
    pltpu.sync_copy(dst_hbm.at[wid], dst_v)
    plsc.subcore_barrier()

    @pl.loop(0, NCHUNK)
    def _(j):
        pltpu.async_copy(x_hbm.at[src_v.at[j]], rows_v, sem).wait()
        pltpu.sync_copy(rows_v, acc.at[dst_v.at[j]], add=True)

    plsc.subcore_barrier()
    pltpu.sync_copy(acc.at[pl.ds(s * RPT, RPT)],
                    out_hbm.at[c, pl.ds(s * RPT, RPT)])


_agg = functools.partial(
    pl.kernel,
    out_type=jax.ShapeDtypeStruct((NC, NACC, D), jnp.float32),
    mesh=plsc.VectorSubcoreMesh(core_axis_name="c", subcore_axis_name="s"),
    scratch_types=[
        pltpu.VMEM((NCHUNK, CHUNK), jnp.int32),
        pltpu.VMEM((NCHUNK, CHUNK), jnp.int32),
        pltpu.VMEM((CHUNK, D), jnp.float32),
        pltpu.VMEM_SHARED((NACC, D), jnp.float32),
        pltpu.SemaphoreType.DMA,
    ],
)(_agg_body)


def _gin_layer_body(x_ref, p_ref, w1_ref, b1_ref, s_ref, be_ref,
                    w2_ref, b2_ref, o_ref):
    z = x_ref[...] + p_ref[0, :N, :] + p_ref[1, :N, :]
    h = jnp.dot(z, w1_ref[...], preferred_element_type=jnp.float32)
    h = h + b1_ref[...]
    mu = jnp.mean(h, axis=0, keepdims=True)
    var = jnp.mean(jnp.square(h - mu), axis=0, keepdims=True)
    h = (h - mu) * lax.rsqrt(var + 1e-5) * s_ref[...] + be_ref[...]
    h = jnp.maximum(h, 0.0)
    h2 = jnp.dot(h, w2_ref[...], preferred_element_type=jnp.float32)
    o_ref[...] = jnp.maximum(h2 + b2_ref[...], 0.0)


def _final_body(h_ref, q_ref, w1_ref, b1_ref, s_ref, be_ref, w2_ref, b2_ref,
                mw1_ref, mb1_ref, mw2_ref, mb2_ref, o_ref):
    z = h_ref[...] + q_ref[0, :N, :] + q_ref[1, :N, :]
    h = jnp.dot(z, w1_ref[...], preferred_element_type=jnp.float32)
    h = h + b1_ref[...]
    mu = jnp.mean(h, axis=0, keepdims=True)
    var = jnp.mean(jnp.square(h - mu), axis=0, keepdims=True)
    h = (h - mu) * lax.rsqrt(var + 1e-5) * s_ref[...] + be_ref[...]
    h = jnp.maximum(h, 0.0)
    h2 = jnp.dot(h, w2_ref[...], preferred_element_type=jnp.float32)
    h2 = jnp.maximum(h2 + b2_ref[...], 0.0)
    g = jnp.dot(h2, mw1_ref[...], preferred_element_type=jnp.float32)
    g = jnp.maximum(g + mb1_ref[...], 0.0)
    o = jnp.dot(g, mw2_ref[...], preferred_element_type=jnp.float32)
    o_ref[...] = o + mb2_ref[...]


def _gin_layer(x, p, w1, b1, s, be, w2, b2):
    return pl.pallas_call(
        _gin_layer_body,
        out_shape=jax.ShapeDtypeStruct((N, H), jnp.float32),
    )(x, p, w1, b1.reshape(1, H), s.reshape(1, H), be.reshape(1, H),
      w2, b2.reshape(1, H))


def _final(h, q, w1, b1, s, be, w2, b2, mw1, mb1, mw2, mb2):
    return pl.pallas_call(
        _final_body,
        out_shape=jax.ShapeDtypeStruct((N, C), jnp.float32),
    )(h, q, w1, b1.reshape(1, H), s.reshape(1, H), be.reshape(1, H),
      w2, b2.reshape(1, H), mw1, mb1.reshape(1, H), mw2, mb2.reshape(1, C))


@jax.jit
def kernel(x, edge_index, c1_W1, c1_b1, c1_s, c1_be, c1_W2, c1_b2,
           c2_W1, c2_b1, c2_s, c2_be, c2_W2, c2_b2,
           m_W1, m_b1, m_W2, m_b2):
    ei = edge_index.astype(jnp.int32)
    pad = EPAD - E
    src = jnp.concatenate([ei[0], jnp.zeros((pad,), jnp.int32)])
    dst = jnp.concatenate([ei[1], jnp.full((pad,), N, jnp.int32)])
    src3 = src.reshape(NW, NCHUNK, CHUNK)
    dst3 = dst.reshape(NW, NCHUNK, CHUNK)
    zero = jnp.zeros((RPT, D), jnp.float32)

    p = _agg(x, src3, dst3, zero)
    h = _gin_layer(x, p, c1_W1, c1_b1, c1_s, c1_be, c1_W2, c1_b2)
    q = _agg(h, src3, dst3, zero)
    return _final(h, q, c2_W1, c2_b1, c2_s, c2_be, c2_W2, c2_b2,
                  m_W1, m_b1, m_W2, m_b2)
